# Initial kernel scaffold; baseline (speedup 1.0000x reference)
import jax
import jax.numpy as jnp
from jax.experimental import pallas as pl


def kernel(output, target):
    raise NotImplementedError("write your pallas kernel here")


# speedup vs baseline: 37.0688x; 37.0688x over previous
import functools

import jax
import jax.numpy as jnp
from jax import lax
from jax.experimental import pallas as pl

_ANCHORS = [1.3221, 1.73145, 3.19275, 4.00944, 5.05587, 8.09892, 9.47112,
            4.84053, 11.2364, 10.0071]
_AW = _ANCHORS[0::2]
_AH = _ANCHORS[1::2]
_NA = 5
_NC = 20
_NH = 19
_NW = 19
_NP = _NH * _NW
_NT = 50
_OBJ_SCALE = 5.0
_SIL = 0.6


def _loss_body(out_ref, tgt_ref, acc_ref):
    b = pl.program_id(0)
    blk = out_ref[0]
    tg = tgt_ref[0]

    f32 = jnp.float32
    t0 = tg[:, 0:1]
    t1 = tg[:, 1:2]
    gx = t1 * float(_NW)
    gy = tg[:, 2:3] * float(_NH)
    gw = tg[:, 3:4] * float(_NW)
    gh = tg[:, 4:5] * float(_NH)

    z = jnp.where(t1 == 0.0, 1.0, 0.0)
    zT = jnp.transpose(z)
    r_i = lax.broadcasted_iota(jnp.int32, (_NT, _NT), 0)
    c_i = lax.broadcasted_iota(jnp.int32, (_NT, _NT), 1)
    badcnt = jnp.sum(jnp.where(c_i <= r_i, zT + jnp.zeros((_NT, _NT), f32), 0.0),
                     axis=1, keepdims=True)
    valid = badcnt == 0.0
    validf = jnp.where(valid, 1.0, 0.0)

    def _anchor_iou(a):
        cwa = jnp.minimum(_AW[a], gw)
        cha = jnp.minimum(_AH[a], gh)
        inter_a = cwa * cha
        return jnp.where((cwa <= 0.0) | (cha <= 0.0), 0.0,
                         inter_a / (_AW[a] * _AH[a] + gw * gh - inter_a))

    best_val = _anchor_iou(0)
    best_idx = jnp.zeros((_NT, 1), jnp.int32)
    for a in range(1, _NA):
        cand = _anchor_iou(a)
        m = cand > best_val
        best_idx = jnp.where(m, a, best_idx)
        best_val = jnp.maximum(best_val, cand)
    n_w = jnp.where(best_val > 0.0, best_idx, _NA - 1)

    gi = gx.astype(jnp.int32)
    gj = gy.astype(jnp.int32)
    pidx = gj * _NW + gi
    tx_val = gx - gi.astype(f32)
    ty_val = gy - gj.astype(f32)

    aw_sel = jnp.full((_NT, 1), _AW[0], f32)
    ah_sel = jnp.full((_NT, 1), _AH[0], f32)
    for a in range(1, _NA):
        aw_sel = jnp.where(n_w == a, _AW[a], aw_sel)
        ah_sel = jnp.where(n_w == a, _AH[a], ah_sel)
    tw_val = jnp.log(gw / aw_sel)
    th_val = jnp.log(gh / ah_sel)
    clsidx = t0.astype(jnp.int32)

    cellid = (n_w * _NP + pidx).astype(f32)
    cellT = jnp.transpose(cellid)
    validT = jnp.transpose(validf)
    conflict = jnp.where((c_i > r_i) & (cellT == cellid) & (validT > 0.0),
                         1.0, 0.0)
    winner = valid & (jnp.sum(conflict, axis=1, keepdims=True) == 0.0)

    lane_p = lax.broadcasted_iota(jnp.int32, (1, _NP), 1)
    gridx = (lane_p % _NW).astype(f32)
    gridy = (lane_p // _NW).astype(f32)

    dense = jnp.zeros((), f32)
    tconf_val = jnp.zeros((_NT, 1), f32)
    curmax_cell = jnp.zeros((_NT, 1), f32)
    g_sx = jnp.zeros((_NT, 1), f32)
    g_sy = jnp.zeros((_NT, 1), f32)
    g_w = jnp.zeros((_NT, 1), f32)
    g_h = jnp.zeros((_NT, 1), f32)
    g_sc = jnp.zeros((_NT, 1), f32)
    glog = jnp.zeros((_NT, _NC), f32)

    for a in range(_NA):
        base = a * (5 + _NC)
        xr = blk[base + 0:base + 1, :]
        yr = blk[base + 1:base + 2, :]
        wr = blk[base + 2:base + 3, :]
        hr = blk[base + 3:base + 4, :]
        cr = blk[base + 4:base + 5, :]
        sx = jax.nn.sigmoid(xr)
        sy = jax.nn.sigmoid(yr)
        sc = jax.nn.sigmoid(cr)
        dense = dense + 0.5 * jnp.sum((sx - 0.5) ** 2 + (sy - 0.5) ** 2
                                      + wr * wr + hr * hr)

        px = sx + gridx
        py = sy + gridy
        pw = jnp.exp(wr) * _AW[a]
        ph = jnp.exp(hr) * _AH[a]
        uw = jnp.maximum(px + pw * 0.5, gx + gw * 0.5) - \
            jnp.minimum(px - pw * 0.5, gx - gw * 0.5)
        uh = jnp.maximum(py + ph * 0.5, gy + gh * 0.5) - \
            jnp.minimum(py - ph * 0.5, gy - gh * 0.5)
        cw = pw + gw - uw
        chh = ph + gh - uh
        inter = jnp.where((cw <= 0.0) | (chh <= 0.0), 0.0, cw * chh)
        iou = inter / (pw * ph + gw * gh - inter)
        curmax_a = jnp.max(iou * validf, axis=0, keepdims=True)
        cmb_a = jnp.where(curmax_a > _SIL, 0.0, 1.0)
        dense = dense + 0.5 * jnp.sum(cmb_a * sc * sc)

        maskA = jnp.where((lane_p == pidx) & (n_w == a), 1.0, 0.0)
        tconf_val = tconf_val + jnp.sum(maskA * iou, axis=1, keepdims=True)
        curmax_cell = curmax_cell + jnp.sum(maskA * curmax_a, axis=1,
                                            keepdims=True)
        g_sx = g_sx + jnp.sum(maskA * sx, axis=1, keepdims=True)
        g_sy = g_sy + jnp.sum(maskA * sy, axis=1, keepdims=True)
        g_w = g_w + jnp.sum(maskA * wr, axis=1, keepdims=True)
        g_h = g_h + jnp.sum(maskA * hr, axis=1, keepdims=True)
        g_sc = g_sc + jnp.sum(maskA * sc, axis=1, keepdims=True)
        clsA = blk[base + 5:base + 5 + _NC, :]
        glog = glog + lax.dot_general(
            maskA, clsA, (((1,), (1,)), ((), ())),
            preferred_element_type=f32)

    cmb_cell = jnp.where(curmax_cell > _SIL, 0.0, 1.0)
    gmax = jnp.max(glog, axis=1, keepdims=True)
    lse = jnp.log(jnp.sum(jnp.exp(glog - gmax), axis=1, keepdims=True)) + gmax
    lane_c = lax.broadcasted_iota(jnp.int32, (_NT, _NC), 1)
    picked = jnp.sum(jnp.where(lane_c == clsidx, glog, 0.0),
                     axis=1, keepdims=True)
    dcls = lse - picked

    delta = (0.5 * ((g_sx - tx_val) ** 2 - (g_sx - 0.5) ** 2)
             + 0.5 * ((g_sy - ty_val) ** 2 - (g_sy - 0.5) ** 2)
             + 0.5 * ((g_w - tw_val) ** 2 - g_w * g_w)
             + 0.5 * ((g_h - th_val) ** 2 - g_h * g_h)
             + 0.5 * (_OBJ_SCALE * (g_sc - tconf_val) ** 2
                      - cmb_cell * g_sc * g_sc)
             + dcls)
    sparse = jnp.sum(jnp.where(winner, delta, 0.0))

    @pl.when(b == 0)
    def _():
        acc_ref[:, :] = jnp.zeros((1, 1), f32)

    acc_ref[:, :] += jnp.reshape(dense + sparse, (1, 1))


@jax.jit
def kernel(output, target):
    nB = output.shape[0]
    outp = output.reshape(nB, _NA * (5 + _NC), _NP)
    tgt = target.reshape(nB, _NT, 5)
    res = pl.pallas_call(
        _loss_body,
        grid=(nB,),
        in_specs=[
            pl.BlockSpec((1, _NA * (5 + _NC), _NP), lambda b: (b, 0, 0)),
            pl.BlockSpec((1, _NT, 5), lambda b: (b, 0, 0)),
        ],
        out_specs=pl.BlockSpec((1, 1), lambda b: (0, 0)),
        out_shape=jax.ShapeDtypeStruct((1, 1), jnp.float32),
    )(outp, tgt)
    return res[0, 0]
